# baseline (speedup 1.0000x reference)
import functools

import jax
import jax.numpy as jnp
from jax import lax
from jax.experimental import pallas as pl
from jax.experimental.pallas import tpu as pltpu
from jax.experimental.pallas import tpu_sc as plsc

N_USERS = 50000
DF = 256
BB = 4096
DEG = 32

NC, NS = 2, 16
NW = NC * NS
ROWS_W = BB // NW
EDGES_W = ROWS_W * DEG
ECHUNK = 128
RCHUNK = ECHUNK // DEG
NCHUNK = EDGES_W // ECHUNK
NLV = DF // 16


DFI = DF // 2


def _sc_direct(uid, pos, neg, user_emb, movie_i):
    mesh = plsc.VectorSubcoreMesh(core_axis_name="c", subcore_axis_name="s")
    out_type = [jax.ShapeDtypeStruct((BB, DF), jnp.float32),
                jax.ShapeDtypeStruct((BB, DFI), jnp.int32),
                jax.ShapeDtypeStruct((BB, DFI), jnp.int32)]
    scratch = [
        pltpu.VMEM((3, ROWS_W), jnp.int32),
        pltpu.VMEM((ROWS_W, DF), jnp.float32),
        pltpu.VMEM((2, ROWS_W, DFI), jnp.int32),
        pltpu.SemaphoreType.DMA,
        pltpu.SemaphoreType.DMA,
        pltpu.SemaphoreType.DMA,
    ]

    @functools.partial(pl.kernel, mesh=mesh, out_type=out_type,
                       scratch_types=scratch)
    def k(uid_h, pos_h, neg_h, uemb_h, memb_h,
          xu_h, xp_h, xn_h, gidx_v, buf_v, bufb_v, sem0, sem1, sem2):
        wid = lax.axis_index("s") * NC + lax.axis_index("c")
        base = pl.multiple_of(wid * ROWS_W, ROWS_W)
        for t, ids_h in enumerate((uid_h, pos_h, neg_h)):
            pltpu.sync_copy(ids_h.at[pl.ds(base, ROWS_W)], gidx_v.at[t])
        pltpu.async_copy(uemb_h.at[gidx_v.at[0]], buf_v, sem0)
        pltpu.async_copy(memb_h.at[gidx_v.at[1]], bufb_v.at[0], sem1)
        pltpu.async_copy(memb_h.at[gidx_v.at[2]], bufb_v.at[1], sem2)
        pltpu.make_async_copy(uemb_h.at[pl.ds(0, ROWS_W)],
                              buf_v, sem0).wait()
        pltpu.sync_copy(buf_v, xu_h.at[pl.ds(base, ROWS_W)])
        pltpu.make_async_copy(memb_h.at[pl.ds(0, ROWS_W)],
                              bufb_v.at[0], sem1).wait()
        pltpu.sync_copy(bufb_v.at[0], xp_h.at[pl.ds(base, ROWS_W)])
        pltpu.make_async_copy(memb_h.at[pl.ds(0, ROWS_W)],
                              bufb_v.at[1], sem2).wait()
        pltpu.sync_copy(bufb_v.at[1], xn_h.at[pl.ds(base, ROWS_W)])

    return k(uid, pos, neg, user_emb, movie_i)


NBV = DF // 32
_ILV = plsc.PackFormat.INTERLEAVED


def _sc_edges(esrc, movie_i):
    mesh = plsc.VectorSubcoreMesh(core_axis_name="c", subcore_axis_name="s")
    out_type = jax.ShapeDtypeStruct((BB, DF), jnp.float32)
    scratch = [
        pltpu.VMEM((EDGES_W,), jnp.int32),
        pltpu.VMEM((2, ECHUNK, DFI), jnp.int32),
        pltpu.VMEM((ROWS_W, DF), jnp.float32),
        pltpu.SemaphoreType.DMA,
        pltpu.SemaphoreType.DMA,
    ]

    @functools.partial(pl.kernel, mesh=mesh, out_type=out_type,
                       scratch_types=scratch)
    def k(esrc_h, memb_h, s_h, eidx_v, ebuf_v, sbuf_v, sem0, sem1):
        wid = lax.axis_index("s") * NC + lax.axis_index("c")
        base = pl.multiple_of(wid * ROWS_W, ROWS_W)
        ebase = pl.multiple_of(wid * EDGES_W, EDGES_W)
        sems = (sem0, sem1)

        pltpu.sync_copy(esrc_h.at[pl.ds(ebase, EDGES_W)], eidx_v)

        def fire(c, par):
            off = pl.multiple_of(c * ECHUNK, ECHUNK)
            pltpu.async_copy(memb_h.at[eidx_v.at[pl.ds(off, ECHUNK)]],
                             ebuf_v.at[par], sems[par])

        fire(0, 0)
        fire(1, 1)

        himask = jnp.int32(-65536)

        def halves(par, row, j):
            w = ebuf_v[par, row, pl.ds(j * 16, 16)]
            lo = lax.bitcast_convert_type(w << 16, jnp.float32)
            hi = lax.bitcast_convert_type(w & himask, jnp.float32)
            return lo, hi

        def chunk_body(h, carry):
            for par in (0, 1):
                c = 2 * h + par
                pltpu.make_async_copy(memb_h.at[pl.ds(0, ECHUNK)],
                                      ebuf_v.at[par], sems[par]).wait()
                for r in range(RCHUNK):
                    first = [halves(par, r * DEG, j) for j in range(NBV)]
                    acc0 = tuple(h0 for pair in first for h0 in pair)

                    def esum(e, acc):
                        outs = list(acc)
                        for j in range(NBV):
                            a, b = halves(par, r * DEG + e, j)
                            outs[2 * j] = outs[2 * j] + a
                            outs[2 * j + 1] = outs[2 * j + 1] + b
                        return tuple(outs)

                    acc = lax.fori_loop(1, DEG, esum, acc0)
                    row = c * RCHUNK + r
                    for j in range(NBV):
                        sbuf_v[row, pl.ds(j * 16, 16)] = acc[2 * j]
                        sbuf_v[row, pl.ds(DFI + j * 16, 16)] = acc[2 * j + 1]

                @pl.when(c + 2 < NCHUNK)
                def _():
                    fire(c + 2, par)

            return carry

        lax.fori_loop(0, NCHUNK // 2, chunk_body, 0)
        pltpu.sync_copy(sbuf_v, s_h.at[pl.ds(base, ROWS_W)])

    return k(esrc, movie_i)


IBA = 512
IBB = 512
BF = jnp.bfloat16


def _mlp_chain(x0, w_refs, b_refs):
    x, acc = x0, None
    for w_ref, b_ref in zip(w_refs, b_refs):
        x = jnp.maximum(
            jnp.dot(x.astype(BF), w_ref[...].astype(BF),
                    preferred_element_type=jnp.float32) + b_ref[...], 0.0)
        acc = x if acc is None else acc + x
    return acc


def _tca_body(xu_ref, xp_ref, xn_ref, wu_ref, bu_ref, wm_ref, bm_ref,
              w1_ref, b1_ref, w2_ref, b2_ref, w3_ref, b3_ref,
              u0_ref, op_ref, on_ref):
    bm = bm_ref[...]
    wm = wm_ref[...].astype(BF)
    u0_ref[...] = jnp.dot(xu_ref[...].astype(BF), wu_ref[...].astype(BF),
                          preferred_element_type=jnp.float32) + bu_ref[...]
    p0 = jnp.dot(xp_ref[...].astype(BF), wm,
                 preferred_element_type=jnp.float32) + bm
    n0 = jnp.dot(xn_ref[...].astype(BF), wm,
                 preferred_element_type=jnp.float32) + bm
    w_refs = (w1_ref, w2_ref, w3_ref)
    b_refs = (b1_ref, b2_ref, b3_ref)
    op_ref[...] = (p0 + _mlp_chain(p0, w_refs, b_refs)) * 0.25
    on_ref[...] = (n0 + _mlp_chain(n0, w_refs, b_refs)) * 0.25


def _tcb_body(uc_ref, ur_ref, s_ref, u0_ref, wm_ref, bm_ref,
              w1_ref, b1_ref, w2_ref, b2_ref, w3_ref, b3_ref, ou_ref):
    p = (uc_ref[...] == ur_ref[...]).astype(BF)
    comb = jnp.dot(p, s_ref[...].astype(BF),
                   preferred_element_type=jnp.float32)
    ones = jnp.ones((BB, 128), dtype=BF)
    cnt = jnp.dot(p, ones, preferred_element_type=jnp.float32)[:, :1] * DEG
    user_h = jnp.dot((comb / cnt).astype(BF), wm_ref[...].astype(BF),
                     preferred_element_type=jnp.float32) + bm_ref[...]
    chain = _mlp_chain(user_h, (w1_ref, w2_ref, w3_ref),
                       (b1_ref, b2_ref, b3_ref))
    ou_ref[...] = (u0_ref[...] + chain) * 0.25


def _w_specs(n):
    fix = lambda i: (0, 0)
    specs = []
    for _ in range(n):
        specs.append(pl.BlockSpec((DF, DF), fix))
        specs.append(pl.BlockSpec((1, DF), fix))
    return specs


def _tc_posneg(xu, xp, xn, wut, bu, wmt, bm, w1t, b1, w2t, b2, w3t, b3,
               interpret=False):
    blk = lambda i: (i, 0)
    row_spec = pl.BlockSpec((IBA, DF), blk)
    return pl.pallas_call(
        _tca_body,
        grid=(BB // IBA,),
        in_specs=[row_spec, row_spec, row_spec] + _w_specs(5),
        out_specs=[row_spec, row_spec, row_spec],
        out_shape=[jax.ShapeDtypeStruct((BB, DF), jnp.float32)] * 3,
        interpret=interpret,
    )(xu, xp, xn, wut, bu, wmt, bm, w1t, b1, w2t, b2, w3t, b3)


def _tc_user(uc, ur, s, u0, wmt, bm, w1t, b1, w2t, b2, w3t, b3,
             interpret=False):
    blk = lambda i: (i, 0)
    fix = lambda i: (0, 0)
    row_spec = pl.BlockSpec((IBB, DF), blk)
    return pl.pallas_call(
        _tcb_body,
        grid=(BB // IBB,),
        in_specs=[
            pl.BlockSpec((IBB, 1), blk),
            pl.BlockSpec((1, BB), fix),
            pl.BlockSpec((BB, DF), fix),
            row_spec,
        ] + _w_specs(4),
        out_specs=row_spec,
        out_shape=jax.ShapeDtypeStruct((BB, DF), jnp.float32),
        interpret=interpret,
    )(uc, ur, s, u0, wmt, bm, w1t, b1, w2t, b2, w3t, b3)


def kernel(user_ids, pos_movie_ids, neg_movie_ids, source, target,
           user_emb, movie_emb, Wu, bu, Wm, bm, W1, b1, W2, b2, W3, b3):
    del target
    esrc = (source - N_USERS).astype(jnp.int32)
    movie_bf = movie_emb.astype(jnp.bfloat16)
    movie_i = lax.bitcast_convert_type(
        movie_bf.reshape(-1, DFI, 2), jnp.int32)
    xu, xp_i, xn_i = _sc_direct(user_ids.astype(jnp.int32),
                                pos_movie_ids.astype(jnp.int32),
                                neg_movie_ids.astype(jnp.int32),
                                user_emb, movie_i)
    s = _sc_edges(esrc, movie_i)
    unpk = lambda a: lax.bitcast_convert_type(a, jnp.bfloat16).reshape(BB, DF)
    xp, xn = unpk(xp_i), unpk(xn_i)
    bu2, bm2 = bu.reshape(1, DF), bm.reshape(1, DF)
    b12, b22, b32 = b1.reshape(1, DF), b2.reshape(1, DF), b3.reshape(1, DF)
    u0, opos, oneg = _tc_posneg(xu, xp, xn, Wu.T, bu2, Wm.T, bm2,
                                W1.T, b12, W2.T, b22, W3.T, b32)
    uf = user_ids.astype(jnp.float32)
    wmt_deint = jnp.concatenate([Wm.T[0::2], Wm.T[1::2]], axis=0)
    ou = _tc_user(uf.reshape(BB, 1), uf.reshape(1, BB), s, u0,
                  wmt_deint, bm2, W1.T, b12, W2.T, b22, W3.T, b32)
    return (ou, opos, oneg)


# speedup vs baseline: 3.7673x; 3.7673x over previous
import functools

import jax
import jax.numpy as jnp
from jax import lax
from jax.experimental import pallas as pl
from jax.experimental.pallas import tpu as pltpu
from jax.experimental.pallas import tpu_sc as plsc

N_USERS = 50000
DF = 256
BB = 4096
DEG = 32

NC, NS = 2, 16
NW = NC * NS
ROWS_W = BB // NW
EDGES_W = ROWS_W * DEG
ECHUNK = 128
RCHUNK = ECHUNK // DEG
NCHUNK = EDGES_W // ECHUNK
NLV = DF // 16


DFI = DF // 2


def _sc_direct(uid, pos, neg, user_emb, movie_i):
    mesh = plsc.VectorSubcoreMesh(core_axis_name="c", subcore_axis_name="s")
    out_type = [jax.ShapeDtypeStruct((BB, DF), jnp.float32),
                jax.ShapeDtypeStruct((BB, DFI), jnp.int32),
                jax.ShapeDtypeStruct((BB, DFI), jnp.int32)]
    scratch = [
        pltpu.VMEM((3, ROWS_W), jnp.int32),
        pltpu.VMEM((ROWS_W, DF), jnp.float32),
        pltpu.VMEM((2, ROWS_W, DFI), jnp.int32),
        pltpu.SemaphoreType.DMA,
        pltpu.SemaphoreType.DMA,
        pltpu.SemaphoreType.DMA,
    ]

    @functools.partial(pl.kernel, mesh=mesh, out_type=out_type,
                       scratch_types=scratch)
    def k(uid_h, pos_h, neg_h, uemb_h, memb_h,
          xu_h, xp_h, xn_h, gidx_v, buf_v, bufb_v, sem0, sem1, sem2):
        wid = lax.axis_index("s") * NC + lax.axis_index("c")
        base = pl.multiple_of(wid * ROWS_W, ROWS_W)
        for t, ids_h in enumerate((uid_h, pos_h, neg_h)):
            pltpu.sync_copy(ids_h.at[pl.ds(base, ROWS_W)], gidx_v.at[t])
        pltpu.async_copy(uemb_h.at[gidx_v.at[0]], buf_v, sem0)
        pltpu.async_copy(memb_h.at[gidx_v.at[1]], bufb_v.at[0], sem1)
        pltpu.async_copy(memb_h.at[gidx_v.at[2]], bufb_v.at[1], sem2)
        pltpu.make_async_copy(uemb_h.at[pl.ds(0, ROWS_W)],
                              buf_v, sem0).wait()
        pltpu.sync_copy(buf_v, xu_h.at[pl.ds(base, ROWS_W)])
        pltpu.make_async_copy(memb_h.at[pl.ds(0, ROWS_W)],
                              bufb_v.at[0], sem1).wait()
        pltpu.sync_copy(bufb_v.at[0], xp_h.at[pl.ds(base, ROWS_W)])
        pltpu.make_async_copy(memb_h.at[pl.ds(0, ROWS_W)],
                              bufb_v.at[1], sem2).wait()
        pltpu.sync_copy(bufb_v.at[1], xn_h.at[pl.ds(base, ROWS_W)])

    return k(uid, pos, neg, user_emb, movie_i)


NBV = DF // 32
_ILV = plsc.PackFormat.INTERLEAVED


def _sc_edges(esrc, movie_i):
    mesh = plsc.VectorSubcoreMesh(core_axis_name="c", subcore_axis_name="s")
    out_type = jax.ShapeDtypeStruct((BB, DF), jnp.float32)
    scratch = [
        pltpu.VMEM((EDGES_W,), jnp.int32),
        pltpu.VMEM((2, ECHUNK, DFI), jnp.int32),
        pltpu.VMEM((ROWS_W, DF), jnp.float32),
        pltpu.SemaphoreType.DMA,
        pltpu.SemaphoreType.DMA,
    ]

    @functools.partial(pl.kernel, mesh=mesh, out_type=out_type,
                       scratch_types=scratch)
    def k(esrc_h, memb_h, s_h, eidx_v, ebuf_v, sbuf_v, sem0, sem1):
        wid = lax.axis_index("s") * NC + lax.axis_index("c")
        base = pl.multiple_of(wid * ROWS_W, ROWS_W)
        ebase = pl.multiple_of(wid * EDGES_W, EDGES_W)
        sems = (sem0, sem1)

        pltpu.sync_copy(esrc_h.at[pl.ds(ebase, EDGES_W)], eidx_v)

        def fire(c, par):
            off = pl.multiple_of(c * ECHUNK, ECHUNK)
            pltpu.async_copy(memb_h.at[eidx_v.at[pl.ds(off, ECHUNK)]],
                             ebuf_v.at[par], sems[par])

        fire(0, 0)
        fire(1, 1)

        himask = jnp.int32(-65536)

        def halves(par, row, j):
            w = ebuf_v[par, row, pl.ds(j * 16, 16)]
            lo = lax.bitcast_convert_type(w << 16, jnp.float32)
            hi = lax.bitcast_convert_type(w & himask, jnp.float32)
            return lo, hi

        def chunk_body(h, carry):
            for par in (0, 1):
                c = 2 * h + par
                pltpu.make_async_copy(memb_h.at[pl.ds(0, ECHUNK)],
                                      ebuf_v.at[par], sems[par]).wait()
                for r in range(RCHUNK):
                    first = [halves(par, r * DEG, j) for j in range(NBV)]
                    acc0 = tuple(h0 for pair in first for h0 in pair)

                    def esum(e, acc):
                        outs = list(acc)
                        for j in range(NBV):
                            a, b = halves(par, r * DEG + e, j)
                            outs[2 * j] = outs[2 * j] + a
                            outs[2 * j + 1] = outs[2 * j + 1] + b
                        return tuple(outs)

                    acc = lax.fori_loop(1, DEG, esum, acc0)
                    row = c * RCHUNK + r
                    for j in range(NBV):
                        sbuf_v[row, pl.ds(j * 16, 16)] = acc[2 * j]
                        sbuf_v[row, pl.ds(DFI + j * 16, 16)] = acc[2 * j + 1]

                @pl.when(c + 2 < NCHUNK)
                def _():
                    fire(c + 2, par)

            return carry

        lax.fori_loop(0, NCHUNK // 2, chunk_body, 0)
        pltpu.sync_copy(sbuf_v, s_h.at[pl.ds(base, ROWS_W)])

    return k(esrc, movie_i)


IBA = 512
IBB = 512
BF = jnp.bfloat16


def _mlp_chain(x0, w_refs, b_refs):
    x, acc = x0, None
    for w_ref, b_ref in zip(w_refs, b_refs):
        x = jnp.maximum(
            jnp.dot(x.astype(BF), w_ref[...].astype(BF),
                    preferred_element_type=jnp.float32) + b_ref[...], 0.0)
        acc = x if acc is None else acc + x
    return acc


def _tca_body(xu_ref, xp_ref, xn_ref, wu_ref, bu_ref, wm_ref, bm_ref,
              w1_ref, b1_ref, w2_ref, b2_ref, w3_ref, b3_ref,
              u0_ref, op_ref, on_ref):
    bm = bm_ref[...]
    wm = wm_ref[...].astype(BF)
    u0_ref[...] = jnp.dot(xu_ref[...].astype(BF), wu_ref[...].astype(BF),
                          preferred_element_type=jnp.float32) + bu_ref[...]
    p0 = jnp.dot(xp_ref[...].astype(BF), wm,
                 preferred_element_type=jnp.float32) + bm
    n0 = jnp.dot(xn_ref[...].astype(BF), wm,
                 preferred_element_type=jnp.float32) + bm
    w_refs = (w1_ref, w2_ref, w3_ref)
    b_refs = (b1_ref, b2_ref, b3_ref)
    op_ref[...] = (p0 + _mlp_chain(p0, w_refs, b_refs)) * 0.25
    on_ref[...] = (n0 + _mlp_chain(n0, w_refs, b_refs)) * 0.25


def _tcb_body(uc_ref, ur_ref, s_ref, u0_ref, wm_ref, bm_ref,
              w1_ref, b1_ref, w2_ref, b2_ref, w3_ref, b3_ref, ou_ref):
    p = (uc_ref[...] == ur_ref[...]).astype(BF)
    comb = jnp.dot(p, s_ref[...].astype(BF),
                   preferred_element_type=jnp.float32)
    ones = jnp.ones((BB, 128), dtype=BF)
    cnt = jnp.dot(p, ones, preferred_element_type=jnp.float32)[:, :1] * DEG
    user_h = jnp.dot((comb / cnt).astype(BF), wm_ref[...].astype(BF),
                     preferred_element_type=jnp.float32) + bm_ref[...]
    chain = _mlp_chain(user_h, (w1_ref, w2_ref, w3_ref),
                       (b1_ref, b2_ref, b3_ref))
    ou_ref[...] = (u0_ref[...] + chain) * 0.25


def _w_specs(n):
    fix = lambda i: (0, 0)
    specs = []
    for _ in range(n):
        specs.append(pl.BlockSpec((DF, DF), fix))
        specs.append(pl.BlockSpec((1, DF), fix))
    return specs


def _tc_posneg(xu, xp, xn, wut, bu, wmt, bm, w1t, b1, w2t, b2, w3t, b3,
               interpret=False):
    blk = lambda i: (i, 0)
    row_spec = pl.BlockSpec((IBA, DF), blk)
    return pl.pallas_call(
        _tca_body,
        grid=(BB // IBA,),
        in_specs=[row_spec, row_spec, row_spec] + _w_specs(5),
        out_specs=[row_spec, row_spec, row_spec],
        out_shape=[jax.ShapeDtypeStruct((BB, DF), jnp.float32)] * 3,
        interpret=interpret,
    )(xu, xp, xn, wut, bu, wmt, bm, w1t, b1, w2t, b2, w3t, b3)


def _tc_user(uc, ur, s, u0, wmt, bm, w1t, b1, w2t, b2, w3t, b3,
             interpret=False):
    blk = lambda i: (i, 0)
    fix = lambda i: (0, 0)
    row_spec = pl.BlockSpec((IBB, DF), blk)
    return pl.pallas_call(
        _tcb_body,
        grid=(BB // IBB,),
        in_specs=[
            pl.BlockSpec((IBB, 1), blk),
            pl.BlockSpec((1, BB), fix),
            pl.BlockSpec((BB, DF), fix),
            row_spec,
        ] + _w_specs(4),
        out_specs=row_spec,
        out_shape=jax.ShapeDtypeStruct((BB, DF), jnp.float32),
        interpret=interpret,
    )(uc, ur, s, u0, wmt, bm, w1t, b1, w2t, b2, w3t, b3)


def kernel(user_ids, pos_movie_ids, neg_movie_ids, source, target,
           user_emb, movie_emb, Wu, bu, Wm, bm, W1, b1, W2, b2, W3, b3):
    del target
    esrc = (source - N_USERS).astype(jnp.int32)
    mi = lax.bitcast_convert_type(movie_emb, jnp.int32)
    lo16 = ((mi[:, :DFI] + jnp.int32(0x8000)) >> 16) & jnp.int32(0xFFFF)
    hi16 = (mi[:, DFI:] + jnp.int32(0x8000)) & jnp.int32(-65536)
    movie_i = lo16 | hi16
    xu, xp_i, xn_i = _sc_direct(user_ids.astype(jnp.int32),
                                pos_movie_ids.astype(jnp.int32),
                                neg_movie_ids.astype(jnp.int32),
                                user_emb, movie_i)
    s = _sc_edges(esrc, movie_i)

    def unpk(a):
        lo = lax.bitcast_convert_type(a << 16, jnp.float32)
        hi = lax.bitcast_convert_type(a & jnp.int32(-65536), jnp.float32)
        return jnp.concatenate([lo, hi], axis=1)

    xp, xn = unpk(xp_i), unpk(xn_i)
    bu2, bm2 = bu.reshape(1, DF), bm.reshape(1, DF)
    b12, b22, b32 = b1.reshape(1, DF), b2.reshape(1, DF), b3.reshape(1, DF)
    u0, opos, oneg = _tc_posneg(xu, xp, xn, Wu.T, bu2, Wm.T, bm2,
                                W1.T, b12, W2.T, b22, W3.T, b32)
    uf = user_ids.astype(jnp.float32)
    ou = _tc_user(uf.reshape(BB, 1), uf.reshape(1, BB), s, u0,
                  Wm.T, bm2, W1.T, b12, W2.T, b22, W3.T, b32)
    return (ou, opos, oneg)


# speedup vs baseline: 5.0115x; 1.3303x over previous
import functools

import jax
import jax.numpy as jnp
from jax import lax
from jax.experimental import pallas as pl
from jax.experimental.pallas import tpu as pltpu
from jax.experimental.pallas import tpu_sc as plsc

N_USERS = 50000
DF = 256
BB = 4096
DEG = 32

NC, NS = 2, 16
NW = NC * NS
ROWS_W = BB // NW
EDGES_W = ROWS_W * DEG
ECHUNK = 64
RCHUNK = ECHUNK // DEG
NCHUNK = EDGES_W // ECHUNK
NBUF = 4
NLV = DF // 16


def _sc_direct(uid, pos, neg, user_emb, movie_emb):
    mesh = plsc.VectorSubcoreMesh(core_axis_name="c", subcore_axis_name="s")
    out_type = [jax.ShapeDtypeStruct((BB, DF), jnp.float32)] * 3
    scratch = [
        pltpu.VMEM((3, ROWS_W), jnp.int32),
        pltpu.VMEM((2, ROWS_W, DF), jnp.float32),
        pltpu.SemaphoreType.DMA,
        pltpu.SemaphoreType.DMA,
    ]

    @functools.partial(pl.kernel, mesh=mesh, out_type=out_type,
                       scratch_types=scratch)
    def k(uid_h, pos_h, neg_h, uemb_h, memb_h,
          xu_h, xp_h, xn_h, gidx_v, buf_v, sem0, sem1):
        wid = lax.axis_index("s") * NC + lax.axis_index("c")
        base = pl.multiple_of(wid * ROWS_W, ROWS_W)
        for t, ids_h in enumerate((uid_h, pos_h, neg_h)):
            pltpu.sync_copy(ids_h.at[pl.ds(base, ROWS_W)], gidx_v.at[t])
        pltpu.async_copy(uemb_h.at[gidx_v.at[0]], buf_v.at[0], sem0)
        pltpu.async_copy(memb_h.at[gidx_v.at[1]], buf_v.at[1], sem1)
        pltpu.make_async_copy(memb_h.at[pl.ds(0, ROWS_W)],
                              buf_v.at[0], sem0).wait()
        pltpu.sync_copy(buf_v.at[0], xu_h.at[pl.ds(base, ROWS_W)])
        pltpu.async_copy(memb_h.at[gidx_v.at[2]], buf_v.at[0], sem0)
        pltpu.make_async_copy(memb_h.at[pl.ds(0, ROWS_W)],
                              buf_v.at[1], sem1).wait()
        pltpu.sync_copy(buf_v.at[1], xp_h.at[pl.ds(base, ROWS_W)])
        pltpu.make_async_copy(memb_h.at[pl.ds(0, ROWS_W)],
                              buf_v.at[0], sem0).wait()
        pltpu.sync_copy(buf_v.at[0], xn_h.at[pl.ds(base, ROWS_W)])

    return k(uid, pos, neg, user_emb, movie_emb)


def _sc_edges(esrc, movie_emb):
    mesh = plsc.VectorSubcoreMesh(core_axis_name="c", subcore_axis_name="s")
    out_type = jax.ShapeDtypeStruct((BB, DF), jnp.float32)
    scratch = [
        pltpu.VMEM((EDGES_W,), jnp.int32),
        pltpu.VMEM((NBUF, ECHUNK, DF), jnp.float32),
        pltpu.VMEM((ROWS_W, DF), jnp.float32),
    ] + [pltpu.SemaphoreType.DMA] * NBUF

    @functools.partial(pl.kernel, mesh=mesh, out_type=out_type,
                       scratch_types=scratch)
    def k(esrc_h, memb_h, s_h, eidx_v, ebuf_v, sbuf_v, *sems):
        wid = lax.axis_index("s") * NC + lax.axis_index("c")
        base = pl.multiple_of(wid * ROWS_W, ROWS_W)
        ebase = pl.multiple_of(wid * EDGES_W, EDGES_W)

        pltpu.sync_copy(esrc_h.at[pl.ds(ebase, EDGES_W)], eidx_v)

        def fire(c, par):
            off = pl.multiple_of(c * ECHUNK, ECHUNK)
            pltpu.async_copy(memb_h.at[eidx_v.at[pl.ds(off, ECHUNK)]],
                             ebuf_v.at[par], sems[par])

        for par in range(NBUF):
            fire(par, par)

        def chunk_body(h, carry):
            for par in range(NBUF):
                c = NBUF * h + par
                pltpu.make_async_copy(memb_h.at[pl.ds(0, ECHUNK)],
                                      ebuf_v.at[par], sems[par]).wait()
                for r in range(RCHUNK):
                    acc0 = tuple(ebuf_v[par, r * DEG, pl.ds(j * 16, 16)]
                                 for j in range(NLV))

                    def esum(e, acc):
                        return tuple(
                            acc[j]
                            + ebuf_v[par, r * DEG + e, pl.ds(j * 16, 16)]
                            for j in range(NLV))

                    acc = lax.fori_loop(1, DEG, esum, acc0)
                    row = c * RCHUNK + r
                    for j in range(NLV):
                        sbuf_v[row, pl.ds(j * 16, 16)] = acc[j]

                @pl.when(c + NBUF < NCHUNK)
                def _():
                    fire(c + NBUF, par)

            return carry

        lax.fori_loop(0, NCHUNK // NBUF, chunk_body, 0)
        pltpu.sync_copy(sbuf_v, s_h.at[pl.ds(base, ROWS_W)])

    return k(esrc, movie_emb)


IBA = 512
IBB = 512
BF = jnp.bfloat16


def _mlp_chain(x0, w_refs, b_refs):
    x, acc = x0, None
    for w_ref, b_ref in zip(w_refs, b_refs):
        x = jnp.maximum(
            jnp.dot(x.astype(BF), w_ref[...].astype(BF),
                    preferred_element_type=jnp.float32) + b_ref[...], 0.0)
        acc = x if acc is None else acc + x
    return acc


def _tca_body(xu_ref, xp_ref, xn_ref, wu_ref, bu_ref, wm_ref, bm_ref,
              w1_ref, b1_ref, w2_ref, b2_ref, w3_ref, b3_ref,
              u0_ref, op_ref, on_ref):
    bm = bm_ref[...]
    wm = wm_ref[...].astype(BF)
    u0_ref[...] = jnp.dot(xu_ref[...].astype(BF), wu_ref[...].astype(BF),
                          preferred_element_type=jnp.float32) + bu_ref[...]
    p0 = jnp.dot(xp_ref[...].astype(BF), wm,
                 preferred_element_type=jnp.float32) + bm
    n0 = jnp.dot(xn_ref[...].astype(BF), wm,
                 preferred_element_type=jnp.float32) + bm
    w_refs = (w1_ref, w2_ref, w3_ref)
    b_refs = (b1_ref, b2_ref, b3_ref)
    op_ref[...] = (p0 + _mlp_chain(p0, w_refs, b_refs)) * 0.25
    on_ref[...] = (n0 + _mlp_chain(n0, w_refs, b_refs)) * 0.25


def _tcb_body(uc_ref, ur_ref, s_ref, u0_ref, wm_ref, bm_ref,
              w1_ref, b1_ref, w2_ref, b2_ref, w3_ref, b3_ref, ou_ref):
    p = (uc_ref[...] == ur_ref[...]).astype(BF)
    comb = jnp.dot(p, s_ref[...].astype(BF),
                   preferred_element_type=jnp.float32)
    ones = jnp.ones((BB, 128), dtype=BF)
    cnt = jnp.dot(p, ones, preferred_element_type=jnp.float32)[:, :1] * DEG
    user_h = jnp.dot((comb / cnt).astype(BF), wm_ref[...].astype(BF),
                     preferred_element_type=jnp.float32) + bm_ref[...]
    chain = _mlp_chain(user_h, (w1_ref, w2_ref, w3_ref),
                       (b1_ref, b2_ref, b3_ref))
    ou_ref[...] = (u0_ref[...] + chain) * 0.25


def _w_specs(n):
    fix = lambda i: (0, 0)
    specs = []
    for _ in range(n):
        specs.append(pl.BlockSpec((DF, DF), fix))
        specs.append(pl.BlockSpec((1, DF), fix))
    return specs


def _tc_posneg(xu, xp, xn, wut, bu, wmt, bm, w1t, b1, w2t, b2, w3t, b3,
               interpret=False):
    blk = lambda i: (i, 0)
    row_spec = pl.BlockSpec((IBA, DF), blk)
    return pl.pallas_call(
        _tca_body,
        grid=(BB // IBA,),
        in_specs=[row_spec, row_spec, row_spec] + _w_specs(5),
        out_specs=[row_spec, row_spec, row_spec],
        out_shape=[jax.ShapeDtypeStruct((BB, DF), jnp.float32)] * 3,
        interpret=interpret,
    )(xu, xp, xn, wut, bu, wmt, bm, w1t, b1, w2t, b2, w3t, b3)


def _tc_user(uc, ur, s, u0, wmt, bm, w1t, b1, w2t, b2, w3t, b3,
             interpret=False):
    blk = lambda i: (i, 0)
    fix = lambda i: (0, 0)
    row_spec = pl.BlockSpec((IBB, DF), blk)
    return pl.pallas_call(
        _tcb_body,
        grid=(BB // IBB,),
        in_specs=[
            pl.BlockSpec((IBB, 1), blk),
            pl.BlockSpec((1, BB), fix),
            pl.BlockSpec((BB, DF), fix),
            row_spec,
        ] + _w_specs(4),
        out_specs=row_spec,
        out_shape=jax.ShapeDtypeStruct((BB, DF), jnp.float32),
        interpret=interpret,
    )(uc, ur, s, u0, wmt, bm, w1t, b1, w2t, b2, w3t, b3)


def kernel(user_ids, pos_movie_ids, neg_movie_ids, source, target,
           user_emb, movie_emb, Wu, bu, Wm, bm, W1, b1, W2, b2, W3, b3):
    del target
    esrc = (source - N_USERS).astype(jnp.int32)
    xu, xp, xn = _sc_direct(user_ids.astype(jnp.int32),
                            pos_movie_ids.astype(jnp.int32),
                            neg_movie_ids.astype(jnp.int32),
                            user_emb, movie_emb)
    s = _sc_edges(esrc, movie_emb)
    bu2, bm2 = bu.reshape(1, DF), bm.reshape(1, DF)
    b12, b22, b32 = b1.reshape(1, DF), b2.reshape(1, DF), b3.reshape(1, DF)
    u0, opos, oneg = _tc_posneg(xu, xp, xn, Wu.T, bu2, Wm.T, bm2,
                                W1.T, b12, W2.T, b22, W3.T, b32)
    uf = user_ids.astype(jnp.float32)
    ou = _tc_user(uf.reshape(BB, 1), uf.reshape(1, BB), s, u0,
                  Wm.T, bm2, W1.T, b12, W2.T, b22, W3.T, b32)
    return (ou, opos, oneg)


# speedup vs baseline: 5.0562x; 1.0089x over previous
import functools

import jax
import jax.numpy as jnp
from jax import lax
from jax.experimental import pallas as pl
from jax.experimental.pallas import tpu as pltpu
from jax.experimental.pallas import tpu_sc as plsc

N_USERS = 50000
DF = 256
BB = 4096
DEG = 32

NC, NS = 2, 16
NW = NC * NS
ROWS_W = BB // NW
EDGES_W = ROWS_W * DEG
ECHUNK = 32
RCHUNK = ECHUNK // DEG
NCHUNK = EDGES_W // ECHUNK
NBUF = 8
NLV = DF // 16


def _sc_direct(uid, pos, neg, user_emb, movie_emb):
    mesh = plsc.VectorSubcoreMesh(core_axis_name="c", subcore_axis_name="s")
    out_type = [jax.ShapeDtypeStruct((BB, DF), jnp.float32)] * 3
    scratch = [
        pltpu.VMEM((3, ROWS_W), jnp.int32),
        pltpu.VMEM((2, ROWS_W, DF), jnp.float32),
        pltpu.SemaphoreType.DMA,
        pltpu.SemaphoreType.DMA,
    ]

    @functools.partial(pl.kernel, mesh=mesh, out_type=out_type,
                       scratch_types=scratch)
    def k(uid_h, pos_h, neg_h, uemb_h, memb_h,
          xu_h, xp_h, xn_h, gidx_v, buf_v, sem0, sem1):
        wid = lax.axis_index("s") * NC + lax.axis_index("c")
        base = pl.multiple_of(wid * ROWS_W, ROWS_W)
        for t, ids_h in enumerate((uid_h, pos_h, neg_h)):
            pltpu.sync_copy(ids_h.at[pl.ds(base, ROWS_W)], gidx_v.at[t])
        pltpu.async_copy(uemb_h.at[gidx_v.at[0]], buf_v.at[0], sem0)
        pltpu.async_copy(memb_h.at[gidx_v.at[1]], buf_v.at[1], sem1)
        pltpu.make_async_copy(memb_h.at[pl.ds(0, ROWS_W)],
                              buf_v.at[0], sem0).wait()
        pltpu.sync_copy(buf_v.at[0], xu_h.at[pl.ds(base, ROWS_W)])
        pltpu.async_copy(memb_h.at[gidx_v.at[2]], buf_v.at[0], sem0)
        pltpu.make_async_copy(memb_h.at[pl.ds(0, ROWS_W)],
                              buf_v.at[1], sem1).wait()
        pltpu.sync_copy(buf_v.at[1], xp_h.at[pl.ds(base, ROWS_W)])
        pltpu.make_async_copy(memb_h.at[pl.ds(0, ROWS_W)],
                              buf_v.at[0], sem0).wait()
        pltpu.sync_copy(buf_v.at[0], xn_h.at[pl.ds(base, ROWS_W)])

    return k(uid, pos, neg, user_emb, movie_emb)


def _sc_edges(esrc, movie_emb):
    mesh = plsc.VectorSubcoreMesh(core_axis_name="c", subcore_axis_name="s")
    out_type = jax.ShapeDtypeStruct((BB, DF), jnp.float32)
    scratch = [
        pltpu.VMEM((EDGES_W,), jnp.int32),
        pltpu.VMEM((NBUF, ECHUNK, DF), jnp.float32),
        pltpu.VMEM((ROWS_W, DF), jnp.float32),
    ] + [pltpu.SemaphoreType.DMA] * NBUF

    @functools.partial(pl.kernel, mesh=mesh, out_type=out_type,
                       scratch_types=scratch)
    def k(esrc_h, memb_h, s_h, eidx_v, ebuf_v, sbuf_v, *sems):
        wid = lax.axis_index("s") * NC + lax.axis_index("c")
        base = pl.multiple_of(wid * ROWS_W, ROWS_W)
        ebase = pl.multiple_of(wid * EDGES_W, EDGES_W)

        pltpu.sync_copy(esrc_h.at[pl.ds(ebase, EDGES_W)], eidx_v)

        def fire(c, par):
            off = pl.multiple_of(c * ECHUNK, ECHUNK)
            pltpu.async_copy(memb_h.at[eidx_v.at[pl.ds(off, ECHUNK)]],
                             ebuf_v.at[par], sems[par])

        for par in range(NBUF):
            fire(par, par)

        def chunk_body(h, carry):
            for par in range(NBUF):
                c = NBUF * h + par
                pltpu.make_async_copy(memb_h.at[pl.ds(0, ECHUNK)],
                                      ebuf_v.at[par], sems[par]).wait()
                for r in range(RCHUNK):
                    acc0 = tuple(ebuf_v[par, r * DEG, pl.ds(j * 16, 16)]
                                 for j in range(NLV))

                    def esum(e, acc):
                        return tuple(
                            acc[j]
                            + ebuf_v[par, r * DEG + e, pl.ds(j * 16, 16)]
                            for j in range(NLV))

                    acc = lax.fori_loop(1, DEG, esum, acc0)
                    row = c * RCHUNK + r
                    for j in range(NLV):
                        sbuf_v[row, pl.ds(j * 16, 16)] = acc[j]

                @pl.when(c + NBUF < NCHUNK)
                def _():
                    fire(c + NBUF, par)

            return carry

        lax.fori_loop(0, NCHUNK // NBUF, chunk_body, 0)
        pltpu.sync_copy(sbuf_v, s_h.at[pl.ds(base, ROWS_W)])

    return k(esrc, movie_emb)


IBA = 512
IBB = 512
BF = jnp.bfloat16


def _mlp_chain(x0, w_refs, b_refs):
    x, acc = x0, None
    for w_ref, b_ref in zip(w_refs, b_refs):
        x = jnp.maximum(
            jnp.dot(x.astype(BF), w_ref[...].astype(BF),
                    preferred_element_type=jnp.float32) + b_ref[...], 0.0)
        acc = x if acc is None else acc + x
    return acc


def _tca_body(xu_ref, xp_ref, xn_ref, wu_ref, bu_ref, wm_ref, bm_ref,
              w1_ref, b1_ref, w2_ref, b2_ref, w3_ref, b3_ref,
              u0_ref, op_ref, on_ref):
    bm = bm_ref[...]
    wm = wm_ref[...].astype(BF)
    u0_ref[...] = jnp.dot(xu_ref[...].astype(BF), wu_ref[...].astype(BF),
                          preferred_element_type=jnp.float32) + bu_ref[...]
    p0 = jnp.dot(xp_ref[...].astype(BF), wm,
                 preferred_element_type=jnp.float32) + bm
    n0 = jnp.dot(xn_ref[...].astype(BF), wm,
                 preferred_element_type=jnp.float32) + bm
    w_refs = (w1_ref, w2_ref, w3_ref)
    b_refs = (b1_ref, b2_ref, b3_ref)
    op_ref[...] = (p0 + _mlp_chain(p0, w_refs, b_refs)) * 0.25
    on_ref[...] = (n0 + _mlp_chain(n0, w_refs, b_refs)) * 0.25


def _tcb_body(uc_ref, ur_ref, s_ref, u0_ref, wm_ref, bm_ref,
              w1_ref, b1_ref, w2_ref, b2_ref, w3_ref, b3_ref, ou_ref):
    p = (uc_ref[...] == ur_ref[...]).astype(BF)
    comb = jnp.dot(p, s_ref[...].astype(BF),
                   preferred_element_type=jnp.float32)
    ones = jnp.ones((BB, 128), dtype=BF)
    cnt = jnp.dot(p, ones, preferred_element_type=jnp.float32)[:, :1] * DEG
    user_h = jnp.dot((comb / cnt).astype(BF), wm_ref[...].astype(BF),
                     preferred_element_type=jnp.float32) + bm_ref[...]
    chain = _mlp_chain(user_h, (w1_ref, w2_ref, w3_ref),
                       (b1_ref, b2_ref, b3_ref))
    ou_ref[...] = (u0_ref[...] + chain) * 0.25


def _w_specs(n):
    fix = lambda i: (0, 0)
    specs = []
    for _ in range(n):
        specs.append(pl.BlockSpec((DF, DF), fix))
        specs.append(pl.BlockSpec((1, DF), fix))
    return specs


def _tc_posneg(xu, xp, xn, wut, bu, wmt, bm, w1t, b1, w2t, b2, w3t, b3,
               interpret=False):
    blk = lambda i: (i, 0)
    row_spec = pl.BlockSpec((IBA, DF), blk)
    return pl.pallas_call(
        _tca_body,
        grid=(BB // IBA,),
        in_specs=[row_spec, row_spec, row_spec] + _w_specs(5),
        out_specs=[row_spec, row_spec, row_spec],
        out_shape=[jax.ShapeDtypeStruct((BB, DF), jnp.float32)] * 3,
        interpret=interpret,
    )(xu, xp, xn, wut, bu, wmt, bm, w1t, b1, w2t, b2, w3t, b3)


def _tc_user(uc, ur, s, u0, wmt, bm, w1t, b1, w2t, b2, w3t, b3,
             interpret=False):
    blk = lambda i: (i, 0)
    fix = lambda i: (0, 0)
    row_spec = pl.BlockSpec((IBB, DF), blk)
    return pl.pallas_call(
        _tcb_body,
        grid=(BB // IBB,),
        in_specs=[
            pl.BlockSpec((IBB, 1), blk),
            pl.BlockSpec((1, BB), fix),
            pl.BlockSpec((BB, DF), fix),
            row_spec,
        ] + _w_specs(4),
        out_specs=row_spec,
        out_shape=jax.ShapeDtypeStruct((BB, DF), jnp.float32),
        interpret=interpret,
    )(uc, ur, s, u0, wmt, bm, w1t, b1, w2t, b2, w3t, b3)


def kernel(user_ids, pos_movie_ids, neg_movie_ids, source, target,
           user_emb, movie_emb, Wu, bu, Wm, bm, W1, b1, W2, b2, W3, b3):
    del target
    esrc = (source - N_USERS).astype(jnp.int32)
    xu, xp, xn = _sc_direct(user_ids.astype(jnp.int32),
                            pos_movie_ids.astype(jnp.int32),
                            neg_movie_ids.astype(jnp.int32),
                            user_emb, movie_emb)
    s = _sc_edges(esrc, movie_emb)
    bu2, bm2 = bu.reshape(1, DF), bm.reshape(1, DF)
    b12, b22, b32 = b1.reshape(1, DF), b2.reshape(1, DF), b3.reshape(1, DF)
    u0, opos, oneg = _tc_posneg(xu, xp, xn, Wu.T, bu2, Wm.T, bm2,
                                W1.T, b12, W2.T, b22, W3.T, b32)
    uf = user_ids.astype(jnp.float32)
    ou = _tc_user(uf.reshape(BB, 1), uf.reshape(1, BB), s, u0,
                  Wm.T, bm2, W1.T, b12, W2.T, b22, W3.T, b32)
    return (ou, opos, oneg)


# speedup vs baseline: 5.1029x; 1.0092x over previous
import functools

import jax
import jax.numpy as jnp
from jax import lax
from jax.experimental import pallas as pl
from jax.experimental.pallas import tpu as pltpu
from jax.experimental.pallas import tpu_sc as plsc

N_USERS = 50000
DF = 256
BB = 4096
DEG = 32

NC, NS = 2, 16
NW = NC * NS
ROWS_W = BB // NW
EDGES_W = ROWS_W * DEG
ECHUNK = 32
RCHUNK = ECHUNK // DEG
NCHUNK = EDGES_W // ECHUNK
NBUF = 8
NLV = DF // 16


def _sc_all(uid, pos, neg, esrc, user_emb, movie_emb):
    mesh = plsc.VectorSubcoreMesh(core_axis_name="c", subcore_axis_name="s")
    out_type = [jax.ShapeDtypeStruct((BB, DF), jnp.float32)] * 4
    scratch = [
        pltpu.VMEM((3, ROWS_W), jnp.int32),
        pltpu.VMEM((EDGES_W,), jnp.int32),
        pltpu.VMEM((NBUF * ECHUNK, DF), jnp.float32),
        pltpu.VMEM((ROWS_W, DF), jnp.float32),
    ] + [pltpu.SemaphoreType.DMA] * NBUF

    @functools.partial(pl.kernel, mesh=mesh, out_type=out_type,
                       scratch_types=scratch)
    def k(uid_h, pos_h, neg_h, esrc_h, uemb_h, memb_h,
          xu_h, xp_h, xn_h, s_h, gidx_v, eidx_v, ebuf_v, sbuf_v, *sems):
        wid = lax.axis_index("s") * NC + lax.axis_index("c")
        base = pl.multiple_of(wid * ROWS_W, ROWS_W)
        ebase = pl.multiple_of(wid * EDGES_W, EDGES_W)

        pltpu.sync_copy(esrc_h.at[pl.ds(ebase, EDGES_W)], eidx_v)
        for t, ids_h in enumerate((uid_h, pos_h, neg_h)):
            pltpu.sync_copy(ids_h.at[pl.ds(base, ROWS_W)], gidx_v.at[t])

        def buf(par, nrows=ECHUNK):
            return ebuf_v.at[pl.ds(pl.multiple_of(par * ECHUNK, ECHUNK),
                                   nrows)]

        def fire(c, par):
            off = pl.multiple_of(c * ECHUNK, ECHUNK)
            pltpu.async_copy(memb_h.at[eidx_v.at[pl.ds(off, ECHUNK)]],
                             buf(par), sems[par])

        for par in range(NBUF):
            fire(par, par)

        def chunk_body(h, carry):
            for par in range(NBUF):
                c = NBUF * h + par
                pltpu.make_async_copy(memb_h.at[pl.ds(0, ECHUNK)],
                                      buf(par), sems[par]).wait()
                for r in range(RCHUNK):
                    erow = par * ECHUNK + r * DEG
                    acc0 = tuple(ebuf_v[erow, pl.ds(j * 16, 16)]
                                 for j in range(NLV))

                    def esum(e, acc):
                        return tuple(
                            acc[j] + ebuf_v[erow + e, pl.ds(j * 16, 16)]
                            for j in range(NLV))

                    acc = lax.fori_loop(1, DEG, esum, acc0)
                    row = c * RCHUNK + r
                    for j in range(NLV):
                        sbuf_v[row, pl.ds(j * 16, 16)] = acc[j]

                @pl.when(c + NBUF < NCHUNK)
                def _():
                    fire(c + NBUF, par)

            return carry

        lax.fori_loop(0, NCHUNK // NBUF, chunk_body, 0)

        half = NBUF * ECHUNK // 2
        lo = ebuf_v.at[pl.ds(0, ROWS_W)]
        hi = ebuf_v.at[pl.ds(half, ROWS_W)]
        pltpu.async_copy(uemb_h.at[gidx_v.at[0]], lo, sems[0])
        pltpu.async_copy(memb_h.at[gidx_v.at[1]], hi, sems[1])
        pltpu.sync_copy(sbuf_v, s_h.at[pl.ds(base, ROWS_W)])
        pltpu.make_async_copy(memb_h.at[pl.ds(0, ROWS_W)], lo, sems[0]).wait()
        pltpu.sync_copy(lo, xu_h.at[pl.ds(base, ROWS_W)])
        pltpu.async_copy(memb_h.at[gidx_v.at[2]], lo, sems[0])
        pltpu.make_async_copy(memb_h.at[pl.ds(0, ROWS_W)], hi, sems[1]).wait()
        pltpu.sync_copy(hi, xp_h.at[pl.ds(base, ROWS_W)])
        pltpu.make_async_copy(memb_h.at[pl.ds(0, ROWS_W)], lo, sems[0]).wait()
        pltpu.sync_copy(lo, xn_h.at[pl.ds(base, ROWS_W)])

    return k(uid, pos, neg, esrc, user_emb, movie_emb)


IBA = 512
IBB = 512
BF = jnp.bfloat16


def _mlp_chain(x0, w_refs, b_refs):
    x, acc = x0, None
    for w_ref, b_ref in zip(w_refs, b_refs):
        x = jnp.maximum(
            jnp.dot(x.astype(BF), w_ref[...].astype(BF),
                    preferred_element_type=jnp.float32) + b_ref[...], 0.0)
        acc = x if acc is None else acc + x
    return acc


def _tc_body(uc_ref, ur_ref, xu_ref, xp_ref, xn_ref, s_ref,
             wu_ref, bu_ref, wm_ref, bm_ref, w1_ref, b1_ref,
             w2_ref, b2_ref, w3_ref, b3_ref, ou_ref, op_ref, on_ref):
    p = (uc_ref[...] == ur_ref[...]).astype(BF)
    comb = jnp.dot(p, s_ref[...].astype(BF),
                   preferred_element_type=jnp.float32)
    ones = jnp.ones((BB, 128), dtype=BF)
    cnt = jnp.dot(p, ones, preferred_element_type=jnp.float32)[:, :1] * DEG
    bm = bm_ref[...]
    wm = wm_ref[...].astype(BF)
    user_h = jnp.dot((comb / cnt).astype(BF), wm,
                     preferred_element_type=jnp.float32) + bm
    u0 = jnp.dot(xu_ref[...].astype(BF), wu_ref[...].astype(BF),
                 preferred_element_type=jnp.float32) + bu_ref[...]
    p0 = jnp.dot(xp_ref[...].astype(BF), wm,
                 preferred_element_type=jnp.float32) + bm
    n0 = jnp.dot(xn_ref[...].astype(BF), wm,
                 preferred_element_type=jnp.float32) + bm
    w_refs = (w1_ref, w2_ref, w3_ref)
    b_refs = (b1_ref, b2_ref, b3_ref)
    ou_ref[...] = (u0 + _mlp_chain(user_h, w_refs, b_refs)) * 0.25
    op_ref[...] = (p0 + _mlp_chain(p0, w_refs, b_refs)) * 0.25
    on_ref[...] = (n0 + _mlp_chain(n0, w_refs, b_refs)) * 0.25


def _w_specs(n):
    fix = lambda i: (0, 0)
    specs = []
    for _ in range(n):
        specs.append(pl.BlockSpec((DF, DF), fix))
        specs.append(pl.BlockSpec((1, DF), fix))
    return specs


def _tc_dense(uc, ur, xu, xp, xn, s, wut, bu, wmt, bm, w1t, b1, w2t, b2,
              w3t, b3, interpret=False):
    blk = lambda i: (i, 0)
    fix = lambda i: (0, 0)
    row_spec = pl.BlockSpec((IBB, DF), blk)
    return pl.pallas_call(
        _tc_body,
        grid=(BB // IBB,),
        in_specs=[
            pl.BlockSpec((IBB, 1), blk),
            pl.BlockSpec((1, BB), fix),
            row_spec, row_spec, row_spec,
            pl.BlockSpec((BB, DF), fix),
        ] + _w_specs(5),
        out_specs=[row_spec, row_spec, row_spec],
        out_shape=[jax.ShapeDtypeStruct((BB, DF), jnp.float32)] * 3,
        interpret=interpret,
    )(uc, ur, xu, xp, xn, s, wut, bu, wmt, bm, w1t, b1, w2t, b2, w3t, b3)


def kernel(user_ids, pos_movie_ids, neg_movie_ids, source, target,
           user_emb, movie_emb, Wu, bu, Wm, bm, W1, b1, W2, b2, W3, b3):
    del target
    esrc = (source - N_USERS).astype(jnp.int32)
    xu, xp, xn, s = _sc_all(user_ids.astype(jnp.int32),
                            pos_movie_ids.astype(jnp.int32),
                            neg_movie_ids.astype(jnp.int32),
                            esrc, user_emb, movie_emb)
    bu2, bm2 = bu.reshape(1, DF), bm.reshape(1, DF)
    b12, b22, b32 = b1.reshape(1, DF), b2.reshape(1, DF), b3.reshape(1, DF)
    uf = user_ids.astype(jnp.float32)
    return _tc_dense(uf.reshape(BB, 1), uf.reshape(1, BB), xu, xp, xn, s,
                     Wu.T, bu2, Wm.T, bm2, W1.T, b12, W2.T, b22, W3.T, b32)
